# baseline (speedup 1.0000x reference)
import jax
import jax.numpy as jnp
from jax.experimental import pallas as pl

NUM_TOKENS = 16384
HIDDEN = 2048
NUM_EXPERTS = 64
BLOCK_TOKENS = 1024
NSPLIT = 2


def _top2(scores):
    iota = jax.lax.broadcasted_iota(jnp.int32, scores.shape, 1)
    m1 = jnp.max(scores, axis=1, keepdims=True)
    i1 = jnp.min(jnp.where(scores == m1, iota, NUM_EXPERTS),
                 axis=1, keepdims=True)
    masked = jnp.where(iota == i1, -jnp.inf, scores)
    m2 = jnp.max(masked, axis=1, keepdims=True)
    i2 = jnp.min(jnp.where(masked == m2, iota, NUM_EXPERTS),
                 axis=1, keepdims=True)
    return (jnp.concatenate([m1, m2], axis=1),
            jnp.concatenate([i1, i2], axis=1))


def _router_kernel(x0_ref, x1_ref, wt_ref, vals_ref, idx_ref):
    wt = wt_ref[...]
    for k, x_ref in enumerate((x0_ref, x1_ref)):
        scores = jnp.dot(x_ref[...], wt, preferred_element_type=jnp.float32)
        v, i = _top2(scores)
        vals_ref[pl.ds(k * BLOCK_TOKENS, BLOCK_TOKENS), :] = v
        idx_ref[pl.ds(k * BLOCK_TOKENS, BLOCK_TOKENS), :] = i


@jax.jit
def kernel(x, W):
    grid = (NUM_TOKENS // (NSPLIT * BLOCK_TOKENS),)
    vals, idx = pl.pallas_call(
        _router_kernel,
        grid=grid,
        in_specs=[
            pl.BlockSpec((BLOCK_TOKENS, HIDDEN), lambda i: (NSPLIT * i, 0)),
            pl.BlockSpec((BLOCK_TOKENS, HIDDEN), lambda i: (NSPLIT * i + 1, 0)),
            pl.BlockSpec((HIDDEN, NUM_EXPERTS), lambda i: (0, 0)),
        ],
        out_specs=[
            pl.BlockSpec((NSPLIT * BLOCK_TOKENS, 2), lambda i: (i, 0)),
            pl.BlockSpec((NSPLIT * BLOCK_TOKENS, 2), lambda i: (i, 0)),
        ],
        out_shape=[
            jax.ShapeDtypeStruct((NUM_TOKENS, 2), jnp.float32),
            jax.ShapeDtypeStruct((NUM_TOKENS, 2), jnp.int32),
        ],
    )(x, x, W.T)
    return vals, idx
